# Initial kernel scaffold; baseline (speedup 1.0000x reference)
import jax
import jax.numpy as jnp
from jax.experimental import pallas as pl


def kernel(x, query, prompts, keys):
    raise NotImplementedError("write your pallas kernel here")


# speedup vs baseline: 1.2535x; 1.2535x over previous
import jax
import jax.numpy as jnp
from jax import lax
from jax.experimental import pallas as pl
from jax.experimental.pallas import tpu as pltpu

EMB = 1024
NPOOL = 8192
TOPK = 8
NBLK = 8
ROWS = NPOOL // NBLK

BATCH = 4096
BBLK = 128


def _select_kernel(q_ref, keys_ref, prompts_hbm, sel_ref, sims_ref, sem):
    i = pl.program_id(0)
    kb = keys_ref[...]
    q = q_ref[...]
    kq = jnp.dot(kb, q.T, preferred_element_type=jnp.float32)
    kn = jnp.sqrt(jnp.sum(kb * kb, axis=1, keepdims=True))
    qn = jnp.sqrt(jnp.sum(q * q))
    sims = kq[:, 0] / jnp.maximum(kn[:, 0] * qn, 1e-8)
    sims_ref[i, :] = sims.reshape(1, ROWS)[0, :]

    @pl.when(i == NBLK - 1)
    def _():
        s = sims_ref[...]
        fidx = (lax.broadcasted_iota(jnp.int32, (NBLK, ROWS), 0) * ROWS
                + lax.broadcasted_iota(jnp.int32, (NBLK, ROWS), 1))
        copies = []
        for k in range(TOPK):
            m = jnp.max(s)
            cand = jnp.where(s == m, fidx, jnp.int32(2 ** 30))
            idx = jnp.min(cand)
            s = jnp.where(fidx == idx, -jnp.inf, s)
            c = pltpu.make_async_copy(
                prompts_hbm.at[pl.ds(idx, 1), :],
                sel_ref.at[pl.ds(k, 1), :],
                sem,
            )
            c.start()
            copies.append(c)
        for c in copies:
            c.wait()


def _add_kernel(sel_ref, x_ref, o_ref):
    o_ref[...] = x_ref[...] + sel_ref[...][None, :, :]


@jax.jit
def kernel(x, query, prompts, keys):
    q2 = query.reshape(1, EMB)
    selected = pl.pallas_call(
        _select_kernel,
        grid=(NBLK,),
        in_specs=[
            pl.BlockSpec((1, EMB), lambda i: (0, 0)),
            pl.BlockSpec((ROWS, EMB), lambda i: (i, 0)),
            pl.BlockSpec(memory_space=pl.ANY),
        ],
        out_specs=pl.BlockSpec((TOPK, EMB), lambda i: (0, 0)),
        out_shape=jax.ShapeDtypeStruct((TOPK, EMB), jnp.float32),
        scratch_shapes=[
            pltpu.VMEM((NBLK, ROWS), jnp.float32),
            pltpu.SemaphoreType.DMA,
        ],
    )(q2, keys, prompts)

    out = pl.pallas_call(
        _add_kernel,
        grid=(BATCH // BBLK,),
        in_specs=[
            pl.BlockSpec((TOPK, EMB), lambda b: (0, 0)),
            pl.BlockSpec((BBLK, TOPK, EMB), lambda b: (b, 0, 0)),
        ],
        out_specs=pl.BlockSpec((BBLK, TOPK, EMB), lambda b: (b, 0, 0)),
        out_shape=jax.ShapeDtypeStruct((BATCH, TOPK, EMB), jnp.float32),
    )(selected, x)
    return out


# speedup vs baseline: 1.2738x; 1.0162x over previous
import jax
import jax.numpy as jnp
from jax import lax
from jax.experimental import pallas as pl
from jax.experimental.pallas import tpu as pltpu

EMB = 1024
NPOOL = 8192
TOPK = 8
NBLK = 8
ROWS = NPOOL // NBLK

BATCH = 4096
BBLK = 256


def _select_kernel(q_ref, keys_ref, prompts_hbm, sel_ref, sims_ref, sem):
    i = pl.program_id(0)
    kb = keys_ref[...]
    q = q_ref[...]
    kq = jnp.dot(kb, q.T, preferred_element_type=jnp.float32)
    kn = jnp.sqrt(jnp.sum(kb * kb, axis=1, keepdims=True))
    qn = jnp.sqrt(jnp.sum(q * q))
    sims = kq[:, 0] / jnp.maximum(kn[:, 0] * qn, 1e-8)
    sims_ref[i, :] = sims.reshape(1, ROWS)[0, :]

    @pl.when(i == NBLK - 1)
    def _():
        s = sims_ref[...]
        fidx = (lax.broadcasted_iota(jnp.int32, (NBLK, ROWS), 0) * ROWS
                + lax.broadcasted_iota(jnp.int32, (NBLK, ROWS), 1))
        copies = []
        for k in range(TOPK):
            m = jnp.max(s)
            cand = jnp.where(s == m, fidx, jnp.int32(2 ** 30))
            idx = jnp.min(cand)
            s = jnp.where(fidx == idx, -jnp.inf, s)
            c = pltpu.make_async_copy(
                prompts_hbm.at[pl.ds(idx, 1), :],
                sel_ref.at[pl.ds(k, 1), :],
                sem,
            )
            c.start()
            copies.append(c)
        for c in copies:
            c.wait()


def _add_kernel(sel_ref, x_ref, o_ref):
    o_ref[...] = x_ref[...] + sel_ref[...][None, :, :]


@jax.jit
def kernel(x, query, prompts, keys):
    q2 = query.reshape(1, EMB)
    selected = pl.pallas_call(
        _select_kernel,
        grid=(NBLK,),
        in_specs=[
            pl.BlockSpec((1, EMB), lambda i: (0, 0)),
            pl.BlockSpec((ROWS, EMB), lambda i: (i, 0)),
            pl.BlockSpec(memory_space=pl.ANY),
        ],
        out_specs=pl.BlockSpec((TOPK, EMB), lambda i: (0, 0)),
        out_shape=jax.ShapeDtypeStruct((TOPK, EMB), jnp.float32),
        scratch_shapes=[
            pltpu.VMEM((NBLK, ROWS), jnp.float32),
            pltpu.SemaphoreType.DMA,
        ],
    )(q2, keys, prompts)

    out = pl.pallas_call(
        _add_kernel,
        grid=(BATCH // BBLK,),
        in_specs=[
            pl.BlockSpec((TOPK, EMB), lambda b: (0, 0)),
            pl.BlockSpec((BBLK, TOPK, EMB), lambda b: (b, 0, 0)),
        ],
        out_specs=pl.BlockSpec((BBLK, TOPK, EMB), lambda b: (b, 0, 0)),
        out_shape=jax.ShapeDtypeStruct((BATCH, TOPK, EMB), jnp.float32),
    )(selected, x)
    return out
